# Initial kernel scaffold; baseline (speedup 1.0000x reference)
import jax
import jax.numpy as jnp
from jax.experimental import pallas as pl


def kernel(x, pos_embedding):
    raise NotImplementedError("write your pallas kernel here")


# speedup vs baseline: 2.3130x; 2.3130x over previous
import jax
import jax.numpy as jnp
from jax.experimental import pallas as pl


def _bcast_body(in_ref, out_ref):
    out_ref[...] = jnp.broadcast_to(in_ref[...][None], out_ref.shape)


def kernel(x, pos_embedding):
    B = x.shape[0]
    S, D = pos_embedding.shape
    BS = 512
    return pl.pallas_call(
        _bcast_body,
        grid=(S // BS,),
        in_specs=[pl.BlockSpec((BS, D), lambda i: (i, 0))],
        out_specs=pl.BlockSpec((B, BS, D), lambda i: (0, i, 0)),
        out_shape=jax.ShapeDtypeStruct((B, S, D), pos_embedding.dtype),
    )(pos_embedding)
